# baseline (speedup 1.0000x reference)
import numpy as np

import jax
import jax.numpy as jnp
from jax import lax
from jax.experimental import pallas as pl
from jax.experimental.pallas import tpu as pltpu
from jax.experimental.pallas import tpu_sc as plsc

N = 10000
D = 128
E = 320000
NC = 2
NS = 16
L = 16
DH = D // NC
EPT = E // NS
CH = 128
NCHUNK = -(-EPT // CH)
EPAD = NS * NCHUNK * CH
NP = N + (EPAD - E)
NB = 80
NBLK = N // NB
TAB = 2048

with np.errstate(divide="ignore"):
  _RTAB = (np.arange(TAB, dtype=np.float32) ** np.float32(-0.5)).astype(
      np.float32)


def _body(xh, rowh, colh, rtabh, outh, tblh,
          acc_s, deg_s, s_s, u_s,
          rix, cix, gA, gB, gC, gD, uv, uw, sv, rtab_t,
          semA, semB, semC, semD,
          semSA, semSB, semSC, semSD):
  c = lax.axis_index("c")
  s = lax.axis_index("s")
  cN = (c * N).astype(jnp.int32)

  zero16 = jnp.zeros((L,), jnp.float32)
  one16 = jnp.ones((L,), jnp.float32)

  pltpu.sync_copy(rowh.at[s], rix)
  pltpu.sync_copy(colh.at[s], cix)
  pltpu.sync_copy(rtabh, rtab_t)

  def _zero_rows(ref, nrows):
    def zr(r, carry):
      for j in range(DH // L):
        ref[r, pl.ds(j * L, L)] = zero16
      return carry
    lax.fori_loop(0, nrows, zr, 0)

  _zero_rows(gA, NB)

  def zsv(i, carry):
    sv[pl.ds(i * L, L)] = zero16
    return carry
  lax.fori_loop(0, NB // L, zsv, 0)

  def ouv(i, carry):
    uv[pl.ds(i * L, L)] = one16
    return carry
  lax.fori_loop(0, CH // L, ouv, 0)

  def _for_owned_blocks(fn):
    def blk(b, carry):
      @pl.when(lax.rem(b, NS) == s)
      def _():
        fn(b)
      return carry
    lax.fori_loop(0, NBLK, blk, 0)

  def zshared(b):
    pltpu.sync_copy(gA.at[pl.ds(0, NB)], acc_s.at[pl.ds(b * NB, NB)])
    pltpu.sync_copy(sv, deg_s.at[pl.ds(b * NB, NB)])
    pltpu.sync_copy(sv, s_s.at[pl.ds(b * NB, NB)])
  with jax.named_scope("ph_zero"):
    _for_owned_blocks(zshared)
  plsc.subcore_barrier()

  drain_src = rtabh.at[pl.ds(0, CH)]

  def degk(g, carry):
    k = 2 * g
    for i, ss in ((0, semSA), (1, semSB)):
      @pl.when(g > 0)
      def _():
        pltpu.make_async_copy(drain_src, uv, ss).wait()
      pltpu.async_copy(uv, deg_s.at[cix.at[k + i]], ss, add=True)
    return carry
  with jax.named_scope("ph_deg"):
    lax.fori_loop(0, NCHUNK // 2, degk, 0)
    for k in range(2 * (NCHUNK // 2), NCHUNK):
      pltpu.async_copy(uv, deg_s.at[cix.at[k]], semSA, add=True)
    pltpu.make_async_copy(drain_src, uv, semSA).wait()
    pltpu.make_async_copy(drain_src, uv, semSA).wait()
    pltpu.make_async_copy(drain_src, uv, semSB).wait()
  plsc.subcore_barrier()


  def _scale_rows(get_scale):
    def srow(r, carry):
      sc = get_scale(r)
      for j in range(DH // L):
        gA[r, pl.ds(j * L, L)] = gA[r, pl.ds(j * L, L)] * sc
      return carry
    lax.fori_loop(0, NB, srow, 0)

  def _bcast(ref, r):
    return plsc.load_gather(ref, [jnp.full((L,), r, jnp.int32)])

  def ublk(b):
    pltpu.sync_copy(deg_s.at[pl.ds(b * NB, NB)], sv)
    def urow(i, carry):
      dv = sv[pl.ds(i * L, L)]
      di = jnp.minimum(dv.astype(jnp.int32), TAB - 1)
      t = dv * plsc.load_gather(rtab_t, [di])
      for _ in range(8):
        t = 0.5 * (t + dv / t)
      sv[pl.ds(i * L, L)] = t / dv
      return carry
    lax.fori_loop(0, NB // L, urow, 0)
    pltpu.sync_copy(sv, u_s.at[pl.ds(b * NB, NB)])
    pltpu.sync_copy(xh.at[c, pl.ds(b * NB, NB)], gA.at[pl.ds(0, NB)])
    _scale_rows(lambda r: _bcast(sv, r))
    pltpu.sync_copy(gA.at[pl.ds(0, NB)], tblh.at[pl.ds(cN + b * NB, NB)])
  with jax.named_scope("ph_u"):
    _for_owned_blocks(ublk)
  plsc.subcore_barrier()

  def sk(g, carry):
    k = 2 * g
    for i, (buf, sg, ss) in ((0, (uv, semA, semSA)), (1, (uw, semB, semSB))):
      @pl.when(g > 0)
      def _():
        pltpu.make_async_copy(drain_src, buf, ss).wait()
      pltpu.async_copy(u_s.at[rix.at[k + i]], buf, sg)
    for i, (buf, sg, ss) in ((0, (uv, semA, semSA)), (1, (uw, semB, semSB))):
      pltpu.make_async_copy(drain_src, buf, sg).wait()
      pltpu.async_copy(buf, s_s.at[cix.at[k + i]], ss, add=True)
    return carry
  with jax.named_scope("ph_S"):
    lax.fori_loop(0, NCHUNK // 2, sk, 0)
    for k in range(2 * (NCHUNK // 2), NCHUNK):
      pltpu.make_async_copy(drain_src, uv, semSA).wait()
      pltpu.async_copy(u_s.at[rix.at[k]], uv, semA)
      pltpu.make_async_copy(drain_src, uv, semA).wait()
      pltpu.async_copy(uv, s_s.at[cix.at[k]], semSA, add=True)
    pltpu.make_async_copy(drain_src, uv, semSA).wait()
    pltpu.make_async_copy(drain_src, uw, semSB).wait()

  def adjk(k, carry):
    def a2(i, c2):
      rix[k, pl.ds(i * L, L)] = rix[k, pl.ds(i * L, L)] + cN
      return c2
    lax.fori_loop(0, CH // L, a2, 0)
    return carry
  with jax.named_scope("ph_adj"):
    lax.fori_loop(0, NCHUNK, adjk, 0)

  bufs = lambda: ((gA, semA, semSA), (gB, semB, semSB),
                  (gC, semC, semSC), (gD, semD, semSD))

  def edge_pass():
    NG = NCHUNK // 4
    def ep(g, carry):
      k = 4 * g
      for i, (gb, sg, ss) in enumerate(bufs()):
        @pl.when(g > 0)
        def _():
          pltpu.make_async_copy(tblh.at[pl.ds(0, CH)], gb, ss).wait()
        pltpu.async_copy(tblh.at[rix.at[k + i]], gb, sg)
      for i, (gb, sg, ss) in enumerate(bufs()):
        pltpu.make_async_copy(tblh.at[pl.ds(0, CH)], gb, sg).wait()
        pltpu.async_copy(gb, acc_s.at[cix.at[k + i]], ss, add=True)
      return carry
    lax.fori_loop(0, NG, ep, 0)
    for k in range(4 * NG, NCHUNK):
      pltpu.make_async_copy(tblh.at[pl.ds(0, CH)], gA, semSA).wait()
      dA = pltpu.async_copy(tblh.at[rix.at[k]], gA, semA)
      dA.wait()
      pltpu.async_copy(gA, acc_s.at[cix.at[k]], semSA, add=True)
    for gb, sg, ss in bufs():
      pltpu.make_async_copy(tblh.at[pl.ds(0, CH)], gb, ss).wait()

  with jax.named_scope("ph_edge1"):
    edge_pass()
  plsc.subcore_barrier()

  _zero_rows(gB, NB)

  def hblk(b):
    pltpu.sync_copy(acc_s.at[pl.ds(b * NB, NB)], gA.at[pl.ds(0, NB)])
    pltpu.sync_copy(s_s.at[pl.ds(b * NB, NB)], sv)
    pltpu.sync_copy(u_s.at[pl.ds(b * NB, NB)], uv.at[pl.ds(0, NB)])
    def us_scale(r):
      uu = _bcast(uv, r)
      ss = _bcast(sv, r)
      return jnp.where(ss > 0.0, uu / ss, 0.0)
    _scale_rows(us_scale)
    pltpu.sync_copy(gA.at[pl.ds(0, NB)], tblh.at[pl.ds(cN + b * NB, NB)])
    pltpu.sync_copy(gB.at[pl.ds(0, NB)], acc_s.at[pl.ds(b * NB, NB)])
  with jax.named_scope("ph_hscale"):
    _for_owned_blocks(hblk)
  plsc.subcore_barrier()

  with jax.named_scope("ph_edge2"):
    edge_pass()
  plsc.subcore_barrier()

  def kblk(b):
    pltpu.sync_copy(acc_s.at[pl.ds(b * NB, NB)], gA.at[pl.ds(0, NB)])
    pltpu.sync_copy(s_s.at[pl.ds(b * NB, NB)], sv)
    def inv_s(r):
      ss = _bcast(sv, r)
      return jnp.where(ss > 0.0, 1.0 / ss, 0.0)
    _scale_rows(inv_s)
    pltpu.sync_copy(gA.at[pl.ds(0, NB)], outh.at[c, pl.ds(b * NB, NB)])
  with jax.named_scope("ph_out"):
    _for_owned_blocks(kblk)


_mesh = plsc.VectorSubcoreMesh(
    core_axis_name="c", subcore_axis_name="s", num_cores=NC, num_subcores=NS)

_gcn2 = pl.kernel(
    _body,
    out_type=[
        jax.ShapeDtypeStruct((NC, N, DH), jnp.float32),
        jax.ShapeDtypeStruct((NC * N, DH), jnp.float32),
    ],
    mesh=_mesh,
    compiler_params=pltpu.CompilerParams(
        needs_layout_passes=False, use_tc_tiling_on_sc=False),
    scratch_types=[
        pltpu.VMEM_SHARED((NP, DH), jnp.float32),
        pltpu.VMEM_SHARED((NP,), jnp.float32),
        pltpu.VMEM_SHARED((NP,), jnp.float32),
        pltpu.VMEM_SHARED((NP,), jnp.float32),
        pltpu.VMEM((NCHUNK, CH), jnp.int32),
        pltpu.VMEM((NCHUNK, CH), jnp.int32),
        pltpu.VMEM((CH, DH), jnp.float32),
        pltpu.VMEM((CH, DH), jnp.float32),
        pltpu.VMEM((CH, DH), jnp.float32),
        pltpu.VMEM((CH, DH), jnp.float32),
        pltpu.VMEM((CH,), jnp.float32),
        pltpu.VMEM((CH,), jnp.float32),
        pltpu.VMEM((NB,), jnp.float32),
        pltpu.VMEM((TAB,), jnp.float32),
        pltpu.SemaphoreType.DMA,
        pltpu.SemaphoreType.DMA,
        pltpu.SemaphoreType.DMA,
        pltpu.SemaphoreType.DMA,
        pltpu.SemaphoreType.DMA,
        pltpu.SemaphoreType.DMA,
        pltpu.SemaphoreType.DMA,
        pltpu.SemaphoreType.DMA,
    ],
)


@jax.jit
def kernel(x, edge_index):
  ei = edge_index.astype(jnp.int32)
  npad = EPAD - E
  row = jnp.concatenate([ei[0], jnp.zeros((npad,), jnp.int32)])
  col = jnp.concatenate([ei[1], N + jnp.arange(npad, dtype=jnp.int32)])
  row3 = row.reshape(NS, NCHUNK, CH)
  col3 = col.reshape(NS, NCHUNK, CH)
  xhalves = jnp.stack([x[:, :DH], x[:, DH:]])
  outh, _ = _gcn2(xhalves, row3, col3, jnp.asarray(_RTAB))
  return jnp.concatenate([outh[0], outh[1]], axis=1)


# speedup vs baseline: 1.1022x; 1.1022x over previous
import numpy as np

import jax
import jax.numpy as jnp
from jax import lax
from jax.experimental import pallas as pl
from jax.experimental.pallas import tpu as pltpu
from jax.experimental.pallas import tpu_sc as plsc

N = 10000
D = 128
E = 320000
NC = 2
NS = 16
L = 16
DH = D // NC
EPT = E // NS
CH = 128
NCTOT = -(-E // CH)
NCHUNK = 164
CNTS = [164] * 14 + [124, 92]
EPAD = sum(CNTS) * CH
NP = N + (EPAD - E)
NB = 80
NBLK = N // NB
TAB = 2048

with np.errstate(divide="ignore"):
  _RTAB = (np.arange(TAB, dtype=np.float32) ** np.float32(-0.5)).astype(
      np.float32)


def _body(xh, rowh, colh, rtabh, outh, tblh,
          acc_s, deg_s, s_s, u_s,
          rix, cix, gA, gB, gC, gD, uv, uw, sv, rtab_t,
          semA, semB, semC, semD,
          semSA, semSB, semSC, semSD):
  c = lax.axis_index("c")
  s = lax.axis_index("s")
  cN = (c * N).astype(jnp.int32)
  nch = (NCHUNK - jnp.where(s == 14, NCHUNK - CNTS[14], 0)
         - jnp.where(s == 15, NCHUNK - CNTS[15], 0))

  zero16 = jnp.zeros((L,), jnp.float32)
  one16 = jnp.ones((L,), jnp.float32)

  pltpu.sync_copy(rowh.at[s], rix)
  pltpu.sync_copy(colh.at[s], cix)
  pltpu.sync_copy(rtabh, rtab_t)

  def _zero_rows(ref, nrows):
    def zr(r, carry):
      for j in range(DH // L):
        ref[r, pl.ds(j * L, L)] = zero16
      return carry
    lax.fori_loop(0, nrows, zr, 0)

  _zero_rows(gA, NB)

  def zsv(i, carry):
    sv[pl.ds(i * L, L)] = zero16
    return carry
  lax.fori_loop(0, NB // L, zsv, 0)

  def ouv(i, carry):
    uv[pl.ds(i * L, L)] = one16
    return carry
  lax.fori_loop(0, CH // L, ouv, 0)

  def _for_owned_blocks(fn):
    def blk(b, carry):
      @pl.when(lax.rem(b, NS) == s)
      def _():
        fn(b)
      return carry
    lax.fori_loop(0, NBLK, blk, 0)

  def zshared(b):
    pltpu.sync_copy(gA.at[pl.ds(0, NB)], acc_s.at[pl.ds(b * NB, NB)])
    pltpu.sync_copy(sv, deg_s.at[pl.ds(b * NB, NB)])
    pltpu.sync_copy(sv, s_s.at[pl.ds(b * NB, NB)])
  with jax.named_scope("ph_zero"):
    _for_owned_blocks(zshared)
  plsc.subcore_barrier()

  drain_src = rtabh.at[pl.ds(0, CH)]

  def degk(g, carry):
    k = 2 * g
    for i, ss in ((0, semSA), (1, semSB)):
      @pl.when(g > 0)
      def _():
        pltpu.make_async_copy(drain_src, uv, ss).wait()
      pltpu.async_copy(uv, deg_s.at[cix.at[k + i]], ss, add=True)
    return carry
  with jax.named_scope("ph_deg"):
    lax.fori_loop(0, nch // 2, degk, 0)
    pltpu.make_async_copy(drain_src, uv, semSA).wait()
    pltpu.make_async_copy(drain_src, uv, semSB).wait()
  plsc.subcore_barrier()


  def _scale_rows(get_scale):
    def srow(r, carry):
      sc = get_scale(r)
      for j in range(DH // L):
        gA[r, pl.ds(j * L, L)] = gA[r, pl.ds(j * L, L)] * sc
      return carry
    lax.fori_loop(0, NB, srow, 0)

  def _bcast(ref, r):
    return plsc.load_gather(ref, [jnp.full((L,), r, jnp.int32)])

  def ublk(b):
    pltpu.sync_copy(deg_s.at[pl.ds(b * NB, NB)], sv)
    def urow(i, carry):
      dv = sv[pl.ds(i * L, L)]
      di = jnp.minimum(dv.astype(jnp.int32), TAB - 1)
      t = dv * plsc.load_gather(rtab_t, [di])
      for _ in range(8):
        t = 0.5 * (t + dv / t)
      sv[pl.ds(i * L, L)] = t / dv
      return carry
    lax.fori_loop(0, NB // L, urow, 0)
    pltpu.sync_copy(sv, u_s.at[pl.ds(b * NB, NB)])
    pltpu.sync_copy(xh.at[c, pl.ds(b * NB, NB)], gA.at[pl.ds(0, NB)])
    _scale_rows(lambda r: _bcast(sv, r))
    pltpu.sync_copy(gA.at[pl.ds(0, NB)], tblh.at[pl.ds(cN + b * NB, NB)])
  with jax.named_scope("ph_u"):
    _for_owned_blocks(ublk)
  plsc.subcore_barrier()

  def sk(g, carry):
    k = 2 * g
    for i, (buf, sg, ss) in ((0, (uv, semA, semSA)), (1, (uw, semB, semSB))):
      @pl.when(g > 0)
      def _():
        pltpu.make_async_copy(drain_src, buf, ss).wait()
      pltpu.async_copy(u_s.at[rix.at[k + i]], buf, sg)
    for i, (buf, sg, ss) in ((0, (uv, semA, semSA)), (1, (uw, semB, semSB))):
      pltpu.make_async_copy(drain_src, buf, sg).wait()
      pltpu.async_copy(buf, s_s.at[cix.at[k + i]], ss, add=True)
    return carry
  with jax.named_scope("ph_S"):
    lax.fori_loop(0, nch // 2, sk, 0)
    pltpu.make_async_copy(drain_src, uv, semSA).wait()
    pltpu.make_async_copy(drain_src, uw, semSB).wait()

  def adjk(k, carry):
    def a2(i, c2):
      rix[k, pl.ds(i * L, L)] = rix[k, pl.ds(i * L, L)] + cN
      return c2
    lax.fori_loop(0, CH // L, a2, 0)
    return carry
  with jax.named_scope("ph_adj"):
    lax.fori_loop(0, nch, adjk, 0)

  bufs = lambda: ((gA, semA, semSA), (gB, semB, semSB),
                  (gC, semC, semSC), (gD, semD, semSD))

  def edge_pass():
    def ep(g, carry):
      k = 4 * g
      for i, (gb, sg, ss) in enumerate(bufs()):
        @pl.when(g > 0)
        def _():
          pltpu.make_async_copy(tblh.at[pl.ds(0, CH)], gb, ss).wait()
        pltpu.async_copy(tblh.at[rix.at[k + i]], gb, sg)
      for i, (gb, sg, ss) in enumerate(bufs()):
        pltpu.make_async_copy(tblh.at[pl.ds(0, CH)], gb, sg).wait()
        pltpu.async_copy(gb, acc_s.at[cix.at[k + i]], ss, add=True)
      return carry
    lax.fori_loop(0, nch // 4, ep, 0)
    for gb, sg, ss in bufs():
      pltpu.make_async_copy(tblh.at[pl.ds(0, CH)], gb, ss).wait()

  with jax.named_scope("ph_edge1"):
    edge_pass()
  plsc.subcore_barrier()

  _zero_rows(gB, NB)

  def hblk(b):
    pltpu.sync_copy(acc_s.at[pl.ds(b * NB, NB)], gA.at[pl.ds(0, NB)])
    pltpu.sync_copy(s_s.at[pl.ds(b * NB, NB)], sv)
    pltpu.sync_copy(u_s.at[pl.ds(b * NB, NB)], uv.at[pl.ds(0, NB)])
    def us_scale(r):
      uu = _bcast(uv, r)
      ss = _bcast(sv, r)
      return jnp.where(ss > 0.0, uu / ss, 0.0)
    _scale_rows(us_scale)
    pltpu.sync_copy(gA.at[pl.ds(0, NB)], tblh.at[pl.ds(cN + b * NB, NB)])
    pltpu.sync_copy(gB.at[pl.ds(0, NB)], acc_s.at[pl.ds(b * NB, NB)])
  with jax.named_scope("ph_hscale"):
    _for_owned_blocks(hblk)
  plsc.subcore_barrier()

  with jax.named_scope("ph_edge2"):
    edge_pass()
  plsc.subcore_barrier()

  def kblk(b):
    pltpu.sync_copy(acc_s.at[pl.ds(b * NB, NB)], gA.at[pl.ds(0, NB)])
    pltpu.sync_copy(s_s.at[pl.ds(b * NB, NB)], sv)
    def inv_s(r):
      ss = _bcast(sv, r)
      return jnp.where(ss > 0.0, 1.0 / ss, 0.0)
    _scale_rows(inv_s)
    pltpu.sync_copy(gA.at[pl.ds(0, NB)], outh.at[c, pl.ds(b * NB, NB)])
  with jax.named_scope("ph_out"):
    _for_owned_blocks(kblk)


_mesh = plsc.VectorSubcoreMesh(
    core_axis_name="c", subcore_axis_name="s", num_cores=NC, num_subcores=NS)

_gcn2 = pl.kernel(
    _body,
    out_type=[
        jax.ShapeDtypeStruct((NC, N, DH), jnp.float32),
        jax.ShapeDtypeStruct((NC * N, DH), jnp.float32),
    ],
    mesh=_mesh,
    compiler_params=pltpu.CompilerParams(
        needs_layout_passes=False, use_tc_tiling_on_sc=False),
    scratch_types=[
        pltpu.VMEM_SHARED((NP, DH), jnp.float32),
        pltpu.VMEM_SHARED((NP,), jnp.float32),
        pltpu.VMEM_SHARED((NP,), jnp.float32),
        pltpu.VMEM_SHARED((NP,), jnp.float32),
        pltpu.VMEM((NCHUNK, CH), jnp.int32),
        pltpu.VMEM((NCHUNK, CH), jnp.int32),
        pltpu.VMEM((CH, DH), jnp.float32),
        pltpu.VMEM((CH, DH), jnp.float32),
        pltpu.VMEM((CH, DH), jnp.float32),
        pltpu.VMEM((CH, DH), jnp.float32),
        pltpu.VMEM((CH,), jnp.float32),
        pltpu.VMEM((CH,), jnp.float32),
        pltpu.VMEM((NB,), jnp.float32),
        pltpu.VMEM((TAB,), jnp.float32),
        pltpu.SemaphoreType.DMA,
        pltpu.SemaphoreType.DMA,
        pltpu.SemaphoreType.DMA,
        pltpu.SemaphoreType.DMA,
        pltpu.SemaphoreType.DMA,
        pltpu.SemaphoreType.DMA,
        pltpu.SemaphoreType.DMA,
        pltpu.SemaphoreType.DMA,
    ],
)


@jax.jit
def kernel(x, edge_index):
  ei = edge_index.astype(jnp.int32)
  npad = EPAD - E
  row = jnp.concatenate([ei[0], jnp.zeros((npad,), jnp.int32)])
  col = jnp.concatenate([ei[1], N + jnp.arange(npad, dtype=jnp.int32)])
  chrow = row.reshape(-1, CH)
  chcol = col.reshape(-1, CH)
  starts = np.concatenate([[0], np.cumsum(CNTS)])
  def slabs(ch):
    out = []
    for t in range(NS):
      sl = ch[starts[t]:starts[t + 1]]
      out.append(jnp.pad(sl, ((0, NCHUNK - CNTS[t]), (0, 0))))
    return jnp.stack(out)
  row3 = slabs(chrow)
  col3 = slabs(chcol)
  xhalves = jnp.stack([x[:, :DH], x[:, DH:]])
  outh, _ = _gcn2(xhalves, row3, col3, jnp.asarray(_RTAB))
  return jnp.concatenate([outh[0], outh[1]], axis=1)
